# baseline (speedup 1.0000x reference)
import functools

import jax
import jax.numpy as jnp
from jax.experimental import pallas as pl
from jax.experimental.pallas import tpu as pltpu

TOP_K = 8


def _router_body(n_blocks, x_ref, w_ref, idx_ref, wgt_ref, acc_ref):
    step = pl.program_id(0)

    @pl.when(step > 0)
    def _tail():
        logits = acc_ref[...]
        e_num = logits.shape[0]
        eid = jax.lax.broadcasted_iota(jnp.int32, logits.shape, 0)
        work = logits
        vals, idxs = [], []
        for _ in range(TOP_K):
            mk = jnp.max(work, axis=0, keepdims=True)
            ik = jnp.min(
                jnp.where(work == mk, eid, e_num), axis=0, keepdims=True
            )
            vals.append(mk)
            idxs.append(ik)
            work = jnp.where(eid == ik, -jnp.inf, work)
        v = jnp.concatenate(vals, axis=0)
        i = jnp.concatenate(idxs, axis=0)
        ex = jnp.exp(v - v[0:1])
        wgt = ex / jnp.sum(ex, axis=0, keepdims=True)
        idx_ref[...] = i.T
        wgt_ref[...] = wgt.T

    @pl.when(step < n_blocks)
    def _matmul():
        x = x_ref[...]
        w = w_ref[...]
        acc_ref[...] = jax.lax.dot_general(
            w, x, (((1,), (1,)), ((), ())),
            preferred_element_type=jnp.float32,
        )


@functools.partial(jax.jit, static_argnames=("block_t",))
def _route(flat_tokens, weight, block_t=1024):
    n, h = flat_tokens.shape
    e_num = weight.shape[0]
    n_blocks = n // block_t
    grid = (n_blocks + 1,)
    idx, wgt = pl.pallas_call(
        functools.partial(_router_body, n_blocks),
        grid=grid,
        in_specs=[
            pl.BlockSpec((block_t, h), lambda i: (jnp.minimum(i, n_blocks - 1), 0)),
            pl.BlockSpec((e_num, h), lambda i: (0, 0)),
        ],
        out_specs=[
            pl.BlockSpec((block_t, TOP_K), lambda i: (jnp.maximum(i - 1, 0), 0)),
            pl.BlockSpec((block_t, TOP_K), lambda i: (jnp.maximum(i - 1, 0), 0)),
        ],
        out_shape=[
            jax.ShapeDtypeStruct((n, TOP_K), jnp.int32),
            jax.ShapeDtypeStruct((n, TOP_K), jnp.float32),
        ],
        scratch_shapes=[pltpu.VMEM((e_num, block_t), jnp.float32)],
        compiler_params=pltpu.CompilerParams(
            dimension_semantics=("arbitrary",),
        ),
    )(flat_tokens, weight)
    return idx, wgt


def kernel(hidden_states, weight):
    if hidden_states.ndim == 2:
        hidden_states = hidden_states[:, None, :]
    bsz, seq_len, hd = hidden_states.shape
    flat = hidden_states.reshape(-1, hd)
    return _route(flat, weight, block_t=1024)


# speedup vs baseline: 1.0177x; 1.0177x over previous
import functools

import jax
import jax.numpy as jnp
from jax.experimental import pallas as pl
from jax.experimental.pallas import tpu as pltpu

TOP_K = 8


_CHUNK = 256


def _router_body(x_ref, w_ref, idx_ref, wgt_ref):
    w = w_ref[...]
    t = x_ref.shape[0]
    for c in range(0, t, _CHUNK):
        x = x_ref[c:c + _CHUNK, :]
        logits = jax.lax.dot_general(
            w, x, (((1,), (1,)), ((), ())),
            preferred_element_type=jnp.float32,
        )
        e_num = logits.shape[0]
        eid = jax.lax.broadcasted_iota(jnp.int32, logits.shape, 0)
        work = logits
        vals, idxs = [], []
        for k in range(TOP_K):
            mk = jnp.max(work, axis=0, keepdims=True)
            ik = jnp.min(
                jnp.where(work == mk, eid, e_num), axis=0, keepdims=True
            )
            vals.append(mk)
            idxs.append(ik)
            if k + 1 < TOP_K:
                work = jnp.where(eid == ik, -jnp.inf, work)
        v = jnp.concatenate(vals, axis=0)
        i = jnp.concatenate(idxs, axis=0)
        ex = jnp.exp(v - v[0:1])
        wgt = ex / jnp.sum(ex, axis=0, keepdims=True)
        idx_ref[c:c + _CHUNK, :] = i.T
        wgt_ref[c:c + _CHUNK, :] = wgt.T


@functools.partial(jax.jit, static_argnames=("block_t",))
def _route(flat_tokens, weight, block_t=1024):
    n, h = flat_tokens.shape
    e_num = weight.shape[0]
    grid = (n // block_t,)
    idx, wgt = pl.pallas_call(
        _router_body,
        grid=grid,
        in_specs=[
            pl.BlockSpec((block_t, h), lambda i: (i, 0)),
            pl.BlockSpec((e_num, h), lambda i: (0, 0)),
        ],
        out_specs=[
            pl.BlockSpec((block_t, TOP_K), lambda i: (i, 0)),
            pl.BlockSpec((block_t, TOP_K), lambda i: (i, 0)),
        ],
        out_shape=[
            jax.ShapeDtypeStruct((n, TOP_K), jnp.int32),
            jax.ShapeDtypeStruct((n, TOP_K), jnp.float32),
        ],
        compiler_params=pltpu.CompilerParams(
            dimension_semantics=("parallel",),
        ),
    )(flat_tokens, weight)
    return idx, wgt


def kernel(hidden_states, weight):
    if hidden_states.ndim == 2:
        hidden_states = hidden_states[:, None, :]
    bsz, seq_len, hd = hidden_states.shape
    flat = hidden_states.reshape(-1, hd)
    return _route(flat, weight, block_t=1024)


# speedup vs baseline: 1.0197x; 1.0019x over previous
import functools

import jax
import jax.numpy as jnp
from jax.experimental import pallas as pl
from jax.experimental.pallas import tpu as pltpu

TOP_K = 8


def _router_body(x_ref, w_ref, idx_ref, wgt_ref):
    x = x_ref[...]
    w = w_ref[...]
    logits = jax.lax.dot_general(
        w, x, (((1,), (1,)), ((), ())), preferred_element_type=jnp.float32
    )
    e_num = logits.shape[0]
    eid = jax.lax.broadcasted_iota(jnp.int32, logits.shape, 0)
    work = logits
    vals, idxs = [], []
    for _ in range(TOP_K):
        mk = jnp.max(work, axis=0, keepdims=True)
        ik = jnp.min(jnp.where(work == mk, eid, e_num), axis=0, keepdims=True)
        vals.append(mk)
        idxs.append(ik)
        work = jnp.where(eid == ik, -jnp.inf, work)
    v = jnp.concatenate(vals, axis=0)
    i = jnp.concatenate(idxs, axis=0)
    ex = jnp.exp(v - v[0:1])
    wgt = ex / jnp.sum(ex, axis=0, keepdims=True)
    idx_ref[...] = i.T
    wgt_ref[...] = wgt.T


@functools.partial(jax.jit, static_argnames=("block_t",))
def _route(flat_tokens, weight, block_t=1024):
    n, h = flat_tokens.shape
    e_num = weight.shape[0]
    grid = (n // block_t,)
    idx, wgt = pl.pallas_call(
        _router_body,
        grid=grid,
        in_specs=[
            pl.BlockSpec((block_t, h), lambda i: (i, 0)),
            pl.BlockSpec((e_num, h), lambda i: (0, 0)),
        ],
        out_specs=[
            pl.BlockSpec((block_t, TOP_K), lambda i: (i, 0)),
            pl.BlockSpec((block_t, TOP_K), lambda i: (i, 0)),
        ],
        out_shape=[
            jax.ShapeDtypeStruct((n, TOP_K), jnp.int32),
            jax.ShapeDtypeStruct((n, TOP_K), jnp.float32),
        ],
        compiler_params=pltpu.CompilerParams(
            dimension_semantics=("parallel",),
        ),
    )(flat_tokens, weight)
    return idx, wgt


def kernel(hidden_states, weight):
    if hidden_states.ndim == 2:
        hidden_states = hidden_states[:, None, :]
    bsz, seq_len, hd = hidden_states.shape
    flat = hidden_states.reshape(-1, hd)
    return _route(flat, weight, block_t=1024)
